# baseline (speedup 1.0000x reference)
import jax
import jax.numpy as jnp
from jax import lax
from jax.experimental import pallas as pl
from jax.experimental.pallas import tpu as pltpu
from jax.experimental.pallas import tpu_sc as plsc

B = 16384
D = 768
N_MODALITY = 4
L = 16
NW = 32
B_PER_W = B // NW
CH = 64
NCHUNK = B_PER_W // CH
D_VECS = D // L


def _body(idx_hbm, code_hbm, qtab_hbm, mtab_hbm, out_hbm,
          idx_v, code_v, qbuf, mbuf, qsem, msem):
    wid = lax.axis_index("s") * 2 + lax.axis_index("c")
    wbase = wid * B_PER_W

    for c in range(NCHUNK):
        base = wbase + c * CH
        pltpu.sync_copy(idx_hbm.at[pl.ds(base, CH)], idx_v)
        pltpu.sync_copy(code_hbm.at[pl.ds(base, CH)], code_v)
        qcp = pltpu.async_copy(qtab_hbm.at[idx_v], qbuf, qsem)
        mcp = pltpu.async_copy(mtab_hbm.at[code_v], mbuf, msem)
        qcp.wait()
        mcp.wait()

        def row_body(i, _):
            def col_body(j, _):
                s = j * L
                qbuf[i, pl.ds(s, L)] = qbuf[i, pl.ds(s, L)] + mbuf[i, pl.ds(s, L)]
                return 0

            lax.fori_loop(0, D_VECS, col_body, 0)
            return 0

        lax.fori_loop(0, CH, row_body, 0)
        pltpu.sync_copy(qbuf, out_hbm.at[pl.ds(base, CH)])


@jax.jit
def _run(idx, code, qtab, mtab):
    mesh = plsc.VectorSubcoreMesh(core_axis_name="c", subcore_axis_name="s")
    return pl.kernel(
        _body,
        out_type=jax.ShapeDtypeStruct((B, D), jnp.float32),
        mesh=mesh,
        scratch_types=[
            pltpu.VMEM((CH,), jnp.int32),
            pltpu.VMEM((CH,), jnp.int32),
            pltpu.VMEM((CH, D), jnp.float32),
            pltpu.VMEM((CH, D), jnp.float32),
            pltpu.SemaphoreType.DMA,
            pltpu.SemaphoreType.DMA,
        ],
    )(idx, code, qtab, mtab)


def kernel(input_text, modality_code, query_table, modality_table):
    idx = input_text.astype(jnp.int32)
    code = modality_code.astype(jnp.int32)
    return _run(idx, code, query_table, modality_table)


# speedup vs baseline: 1.1046x; 1.1046x over previous
import jax
import jax.numpy as jnp
from jax import lax
from jax.experimental import pallas as pl
from jax.experimental.pallas import tpu as pltpu
from jax.experimental.pallas import tpu_sc as plsc

B = 16384
D = 768
N_MODALITY = 4
L = 16
NW = 32
B_PER_W = B // NW
CH = 64
NCHUNK = B_PER_W // CH
D_VECS = D // L


def _body(idx_hbm, code_hbm, qtab_hbm, mtab_hbm, out_hbm,
          idx_v, code_v, qbuf, mbuf, qsem, msem):
    wid = lax.axis_index("s") * 2 + lax.axis_index("c")
    wbase = wid * B_PER_W

    for c in range(NCHUNK):
        base = wbase + c * CH
        pltpu.sync_copy(idx_hbm.at[pl.ds(base, CH)], idx_v)
        pltpu.sync_copy(code_hbm.at[pl.ds(base, CH)], code_v)
        qcp = pltpu.async_copy(qtab_hbm.at[idx_v], qbuf, qsem)
        mcp = pltpu.async_copy(mtab_hbm.at[code_v], mbuf, msem)
        qcp.wait()
        mcp.wait()

        def row_body(i, _):
            for j in range(D_VECS):
                s = j * L
                qbuf[i, pl.ds(s, L)] = qbuf[i, pl.ds(s, L)] + mbuf[i, pl.ds(s, L)]
            return 0

        lax.fori_loop(0, CH, row_body, 0)
        pltpu.sync_copy(qbuf, out_hbm.at[pl.ds(base, CH)])


@jax.jit
def _run(idx, code, qtab, mtab):
    mesh = plsc.VectorSubcoreMesh(core_axis_name="c", subcore_axis_name="s")
    return pl.kernel(
        _body,
        out_type=jax.ShapeDtypeStruct((B, D), jnp.float32),
        mesh=mesh,
        scratch_types=[
            pltpu.VMEM((CH,), jnp.int32),
            pltpu.VMEM((CH,), jnp.int32),
            pltpu.VMEM((CH, D), jnp.float32),
            pltpu.VMEM((CH, D), jnp.float32),
            pltpu.SemaphoreType.DMA,
            pltpu.SemaphoreType.DMA,
        ],
    )(idx, code, qtab, mtab)


def kernel(input_text, modality_code, query_table, modality_table):
    idx = input_text.astype(jnp.int32)
    code = modality_code.astype(jnp.int32)
    return _run(idx, code, query_table, modality_table)
